# baseline (speedup 1.0000x reference)
import functools

import jax
import jax.numpy as jnp
from jax import lax
from jax.experimental import pallas as pl
from jax.experimental.pallas import tpu as pltpu
from jax.experimental.pallas import tpu_sc as plsc

N = 50000
E = 800000
G = 8
D = 64
H = 4
OC = D // H
DFF = 256
L = 2
NUM_RB = 4

NC = 2
NS = 16
NW = NC * NS

EPAD = 819200
GB = 1024
PW_G = EPAD // NW
GBG = 256
NCH2 = PW_G // GBG // 2
PW_S = EPAD // NS
NCH_S = PW_S // GB

NWIN = 12544
NROUND = 2
ACCRV = 12672
ACCRX = 6400
ZRV = ACCRV // NS
ZRX = ACCRX // NS

BN = 2000
BE = 2048


def _prologue_body(pa_ref, ba_ref, npa_ref, wpa_ref, wba_ref, bin_ref,
                   wemb_ref, bemb_ref, inp_ref, x0_ref):
    inp_ref[...] = (pa_ref[...] @ wpa_ref[...] + ba_ref[...] @ wba_ref[...]
                    + bin_ref[...])
    x0_ref[...] = npa_ref[...] @ wemb_ref[...] + bemb_ref[...]


def _qkv_body(x_ref, inp_ref, wq_ref, bq_ref, wk_ref, bk_ref, wv_ref, bv_ref,
              xi_ref, q_ref, kv_ref):
    xi = x_ref[...] + inp_ref[...]
    xi_ref[...] = xi
    q_ref[...] = xi @ wq_ref[...] + bq_ref[...]
    kv_ref[...] = jnp.concatenate(
        [xi @ wk_ref[...] + bk_ref[...], xi @ wv_ref[...] + bv_ref[...]],
        axis=1)


def _edge_body(qd_ref, kvs_ref, a_ref, dst_ref, we_ref, ov_ref, ox_ref):
    e = a_ref[...] @ we_ref[...]
    qd = qd_ref[...]
    kvs = kvs_ref[...]
    ke = kvs[:, :D] + e
    ve = kvs[:, D:] + e
    parts = []
    exs = []
    for h in range(H):
        sl = slice(h * OC, (h + 1) * OC)
        alpha = jnp.sum(qd[:, sl] * ke[:, sl], axis=1, keepdims=True) * 0.25
        ex = jnp.exp(alpha)
        parts.append(ve[:, sl] * ex)
        exs.append(ex)
    ov_ref[...] = jnp.concatenate(parts, axis=1)
    ex4 = jnp.concatenate(exs, axis=1)
    odd = (dst_ref[...] % 2) == 1
    zero = jnp.zeros_like(ex4)
    ox_ref[...] = jnp.concatenate(
        [jnp.where(odd, zero, ex4), jnp.where(odd, ex4, zero)], axis=1)


def _layer_norm(x, g, b):
    m = jnp.mean(x, axis=-1, keepdims=True)
    v = jnp.mean((x - m) ** 2, axis=-1, keepdims=True)
    return (x - m) * jax.lax.rsqrt(v + 1e-5) * g + b


def _node_body(xi_ref, accv_ref, den_ref, ws_ref, bs_ref, g1_ref, be1_ref,
               w1_ref, b1_ref, w2_ref, b2_ref, g2_ref, be2_ref, wa_ref,
               ba_ref, xn_ref, logit_ref):
    acc = accv_ref[...]
    dens = den_ref[...]
    aggs = []
    for h in range(H):
        den = dens[:, h:h + 1] + 1e-16
        aggs.append(acc[:, h * OC:(h + 1) * OC] / den)
    agg = jnp.concatenate(aggs, axis=1)
    xi = xi_ref[...]
    x2 = agg + xi @ ws_ref[...] + bs_ref[...]
    y = _layer_norm(xi + x2, g1_ref[...], be1_ref[...])
    z = jax.nn.relu(y @ w1_ref[...] + b1_ref[...]) @ w2_ref[...] + b2_ref[...]
    xn = _layer_norm(y + z, g2_ref[...], be2_ref[...])
    xn_ref[...] = xn
    logit_ref[...] = xn @ wa_ref[...] + ba_ref[...]


def _pool_body(x_ref, b_ref, wc_ref, bc_ref, sums_ref, cnts_ref, val_ref):
    i = pl.program_id(0)

    @pl.when(i == 0)
    def _init():
        sums_ref[...] = jnp.zeros_like(sums_ref)
        cnts_ref[...] = jnp.zeros_like(cnts_ref)

    x = x_ref[...]
    bv = b_ref[...]
    rows_s = []
    rows_c = []
    for g in range(G):
        mask = (bv == g).astype(jnp.float32)
        rows_s.append(jnp.sum(mask * x, axis=0, keepdims=True))
        rows_c.append(jnp.sum(mask, axis=0, keepdims=True) *
                      jnp.ones((1, D), jnp.float32))
    sums_ref[...] += jnp.concatenate(rows_s, axis=0)
    cnts_ref[...] += jnp.concatenate(rows_c, axis=0)

    @pl.when(i == pl.num_programs(0) - 1)
    def _fin():
        pooled = sums_ref[...] / jnp.maximum(cnts_ref[...], 1.0)
        val_ref[...] = pooled @ wc_ref[...] + bc_ref[...]


_MESH = plsc.VectorSubcoreMesh(core_axis_name="c", subcore_axis_name="s",
                               num_cores=NC, num_subcores=NS)


@functools.partial(
    pl.kernel,
    out_type=[jax.ShapeDtypeStruct((EPAD, 2 * D), jnp.float32),
              jax.ShapeDtypeStruct((EPAD, D), jnp.float32)],
    mesh=_MESH,
    compiler_params=pltpu.CompilerParams(use_tc_tiling_on_sc=False),
    scratch_types=[
        pltpu.VMEM((4, 128), jnp.int32),
        pltpu.VMEM((4, 128), jnp.int32),
        pltpu.VMEM((GBG, 2 * D), jnp.float32),
        pltpu.VMEM((GBG, 2 * D), jnp.float32),
        pltpu.VMEM((GBG, D), jnp.float32),
        pltpu.VMEM((GBG, D), jnp.float32),
        pltpu.SemaphoreType.DMA,
        pltpu.SemaphoreType.DMA,
        pltpu.SemaphoreType.DMA,
        pltpu.SemaphoreType.DMA,
        pltpu.SemaphoreType.DMA,
        pltpu.SemaphoreType.DMA,
    ],
)
def _sc_gather(sd_hbm, kv_hbm, q_hbm, kvs_hbm, qd_hbm,
               ib0, ib1, kvb0, kvb1, qdb0, qdb1,
               isem0, isem1, gsem0, gsem1, wsem0, wsem1):
    c = lax.axis_index("c")
    s = lax.axis_index("s")
    wid = c * NS + s
    cbase = wid * (2 * NCH2)

    bufs = ((ib0, kvb0, qdb0, isem0, gsem0, wsem0),
            (ib1, kvb1, qdb1, isem1, gsem1, wsem1))

    for half, (ib, kvb, qdb, isem, gsem, wsem) in enumerate(bufs):
        r0 = pl.multiple_of((cbase + half) * 4, 4)
        pltpu.async_copy(sd_hbm.at[pl.ds(r0, 4)], ib, isem)

    def body(i2, _):
        for half, (ib, kvb, qdb, isem, gsem, wsem) in enumerate(bufs):
            ci = cbase + 2 * i2 + half
            off = pl.multiple_of(ci * GBG, GBG)
            r0 = pl.multiple_of(ci * 4, 4)
            pltpu.make_async_copy(sd_hbm.at[pl.ds(r0, 4)], ib, isem).wait()

            @pl.when(i2 > 0)
            def _wait_writes():
                poff = pl.multiple_of(off - 2 * GBG, GBG)
                pltpu.make_async_copy(
                    kvb, kvs_hbm.at[pl.ds(poff, GBG)], wsem).wait()
                pltpu.make_async_copy(
                    qdb, qd_hbm.at[pl.ds(poff, GBG)], wsem).wait()

            g = [pltpu.async_copy(kv_hbm.at[ib.at[j]],
                                  kvb.at[pl.ds(j * 128, 128)], gsem)
                 for j in range(2)]
            g += [pltpu.async_copy(q_hbm.at[ib.at[2 + j]],
                                   qdb.at[pl.ds(j * 128, 128)], gsem)
                  for j in range(2)]
            for d in g:
                d.wait()

            @pl.when(i2 < NCH2 - 1)
            def _prefetch():
                r2 = pl.multiple_of((ci + 2) * 4, 4)
                pltpu.async_copy(sd_hbm.at[pl.ds(r2, 4)], ib, isem)

            pltpu.async_copy(kvb, kvs_hbm.at[pl.ds(off, GBG)], wsem)
            pltpu.async_copy(qdb, qd_hbm.at[pl.ds(off, GBG)], wsem)
        return 0

    lax.fori_loop(0, NCH2, body, 0)

    for half, (ib, kvb, qdb, isem, gsem, wsem) in enumerate(bufs):
        loff = pl.multiple_of((cbase + 2 * (NCH2 - 1) + half) * GBG, GBG)
        pltpu.make_async_copy(kvb, kvs_hbm.at[pl.ds(loff, GBG)], wsem).wait()
        pltpu.make_async_copy(qdb, qd_hbm.at[pl.ds(loff, GBG)], wsem).wait()


@functools.partial(
    pl.kernel,
    out_type=[jax.ShapeDtypeStruct((2 * NROUND * ACCRV, D), jnp.float32),
              jax.ShapeDtypeStruct((2 * NROUND * ACCRX, 8), jnp.float32)],
    mesh=_MESH,
    compiler_params=pltpu.CompilerParams(use_tc_tiling_on_sc=False),
    scratch_types=[
        pltpu.VMEM((8, 128), jnp.int32),
        pltpu.VMEM((8, 128), jnp.int32),
        pltpu.VMEM((8, 128), jnp.int32),
        pltpu.VMEM((GB, D), jnp.float32),
        pltpu.VMEM((GB, 8), jnp.float32),
        pltpu.VMEM_SHARED((ACCRV, D), jnp.float32),
        pltpu.VMEM_SHARED((ACCRX, 8), jnp.float32),
    ],
)
def _sc_scatter(dst2_hbm, msgv_hbm, msgx_hbm, zerov_hbm, zerox_hbm,
                outv_hbm, outx_hbm,
                didx_v, lidxv_v, lidxx_v, rowsv_v, rowsx_v, accv_sh, accx_sh):
    c = lax.axis_index("c")
    s = lax.axis_index("s")
    lanes = lax.iota(jnp.int32, 16)

    def do_round(r, _):
        w = 2 * r + c
        nbase = w * NWIN

        pltpu.sync_copy(zerov_hbm, accv_sh.at[pl.ds(s * ZRV, ZRV)])
        pltpu.sync_copy(zerox_hbm, accx_sh.at[pl.ds(s * ZRX, ZRX)])
        plsc.subcore_barrier()

        def chunk(i, _):
            off = pl.multiple_of(s * PW_S + i * GB, GB)
            row0 = pl.multiple_of(off // 128, 8)
            pltpu.sync_copy(dst2_hbm.at[pl.ds(row0, 8)], didx_v)
            pltpu.sync_copy(msgv_hbm.at[pl.ds(off, GB)], rowsv_v)
            pltpu.sync_copy(msgx_hbm.at[pl.ds(off, GB)], rowsx_v)
            for j in range(8):
                for t in range(8):
                    dvec = didx_v[j, pl.ds(t * 16, 16)]
                    lvec = dvec - nbase
                    gpos = off + (j * 128 + t * 16) + lanes
                    ok = (lvec >= 0) & (lvec < NWIN) & (gpos < E)
                    tv = NWIN + (dvec & 127)
                    tx = (NWIN // 2) + (dvec & 127)
                    lidxv_v[j, pl.ds(t * 16, 16)] = jnp.where(ok, lvec, tv)
                    lidxx_v[j, pl.ds(t * 16, 16)] = jnp.where(
                        ok, lvec >> 1, tx)
            for j in range(8):
                pltpu.sync_copy(rowsv_v.at[pl.ds(j * 128, 128)],
                                accv_sh.at[lidxv_v.at[j]], add=True)
                pltpu.sync_copy(rowsx_v.at[pl.ds(j * 128, 128)],
                                accx_sh.at[lidxx_v.at[j]], add=True)
            return 0

        lax.fori_loop(0, NCH_S, chunk, 0)

        plsc.subcore_barrier()
        pltpu.sync_copy(accv_sh.at[pl.ds(s * ZRV, ZRV)],
                        outv_hbm.at[pl.ds(w * ACCRV + s * ZRV, ZRV)])
        pltpu.sync_copy(accx_sh.at[pl.ds(s * ZRX, ZRX)],
                        outx_hbm.at[pl.ds(w * ACCRX + s * ZRX, ZRX)])
        plsc.subcore_barrier()
        return 0

    lax.fori_loop(0, NROUND, do_round, 0)


def _tc_call(body, grid, in_specs, out_specs, out_shape, args):
    return pl.pallas_call(
        body,
        grid=grid,
        in_specs=in_specs,
        out_specs=out_specs,
        out_shape=out_shape,
    )(*args)


def _row(x):
    return x.reshape(1, -1)


def kernel(power_alloc, beam_alloc, node_power_attn, edge_power_attn,
           edge_index, ptr, batch, params):
    p = params
    f32 = jnp.float32

    pa = power_alloc.reshape(N, 16)
    ba = beam_alloc.reshape(N, 8)
    npa = node_power_attn.reshape(N, -1)
    ea = edge_power_attn.reshape(E, -1)
    ea_p = jnp.concatenate([ea, jnp.zeros((EPAD - E, 16), f32)], axis=0)
    src_p = jnp.concatenate([edge_index[0], jnp.zeros((EPAD - E,), jnp.int32)])
    dst_p = jnp.concatenate([edge_index[1], jnp.zeros((EPAD - E,), jnp.int32)])
    dst2 = dst_p.reshape(EPAD // 128, 128)
    sd = jnp.concatenate([src_p.reshape(EPAD // GBG, 2, 128),
                          dst_p.reshape(EPAD // GBG, 2, 128)],
                         axis=1).reshape(EPAD // GBG * 4, 128)
    batch2 = batch.reshape(N, 1)
    dstcol = dst_p.reshape(EPAD, 1)
    zerov = jnp.zeros((ZRV, D), f32)
    zerox = jnp.zeros((ZRX, 8), f32)

    w_in = p["W_in"]
    idx_pa = [6 * rb + j for rb in range(NUM_RB) for j in range(4)]
    idx_ba = [6 * rb + 4 + j for rb in range(NUM_RB) for j in range(2)]
    wpa = w_in[jnp.array(idx_pa)]
    wba = w_in[jnp.array(idx_ba)]

    nb = N // BN
    spec_n64 = pl.BlockSpec((BN, D), lambda i: (i, 0))
    spec_w = lambda r, c: pl.BlockSpec((r, c), lambda i: (0, 0))

    inp, x = _tc_call(
        _prologue_body, (nb,),
        [pl.BlockSpec((BN, 16), lambda i: (i, 0)),
         pl.BlockSpec((BN, 8), lambda i: (i, 0)),
         pl.BlockSpec((BN, 16), lambda i: (i, 0)),
         spec_w(16, D), spec_w(8, D), spec_w(1, D),
         spec_w(16, D), spec_w(1, D)],
        [spec_n64, spec_n64],
        [jax.ShapeDtypeStruct((N, D), f32)] * 2,
        (pa, ba, npa, wpa, wba, _row(p["b_in"]), p["W_emb"], _row(p["b_emb"])),
    )

    logit = None
    for l in range(L):
        xi, q_t, kv_t = _tc_call(
            _qkv_body, (nb,),
            [spec_n64, spec_n64,
             spec_w(D, D), spec_w(1, D), spec_w(D, D), spec_w(1, D),
             spec_w(D, D), spec_w(1, D)],
            [spec_n64, spec_n64, pl.BlockSpec((BN, 2 * D), lambda i: (i, 0))],
            [jax.ShapeDtypeStruct((N, D), f32),
             jax.ShapeDtypeStruct((N, D), f32),
             jax.ShapeDtypeStruct((N, 2 * D), f32)],
            (x, inp, p["Wq"][l], _row(p["bq"][l]), p["Wk"][l],
             _row(p["bk"][l]), p["Wv"][l], _row(p["bv"][l])),
        )

        kvs, qd = _sc_gather(sd, kv_t, q_t)

        neb = EPAD // BE
        spec_e64 = pl.BlockSpec((BE, D), lambda i: (i, 0))
        msgv, msgx = _tc_call(
            _edge_body, (neb,),
            [spec_e64, pl.BlockSpec((BE, 2 * D), lambda i: (i, 0)),
             pl.BlockSpec((BE, 16), lambda i: (i, 0)),
             pl.BlockSpec((BE, 1), lambda i: (i, 0)),
             pl.BlockSpec((16, D), lambda i: (0, 0))],
            [spec_e64, pl.BlockSpec((BE, 8), lambda i: (i, 0))],
            [jax.ShapeDtypeStruct((EPAD, D), f32),
             jax.ShapeDtypeStruct((EPAD, 8), f32)],
            (qd, kvs, ea_p, dstcol, p["We"][l]),
        )

        accv_pad, accx_pad = _sc_scatter(dst2, msgv, msgx, zerov, zerox)
        accv = jnp.concatenate(
            [accv_pad[w * ACCRV:w * ACCRV + NWIN] for w in range(2 * NROUND)],
            axis=0)[:N]
        den = jnp.concatenate(
            [accx_pad[w * ACCRX:w * ACCRX + NWIN // 2].reshape(NWIN, 4)
             for w in range(2 * NROUND)], axis=0)[:N]

        x, logit = _tc_call(
            _node_body, (nb,),
            [spec_n64, spec_n64, pl.BlockSpec((BN, 4), lambda i: (i, 0)),
             spec_w(D, D), spec_w(1, D), spec_w(1, D), spec_w(1, D),
             spec_w(D, DFF), spec_w(1, DFF), spec_w(DFF, D), spec_w(1, D),
             spec_w(1, D), spec_w(1, D), spec_w(D, NUM_RB), spec_w(1, NUM_RB)],
            [spec_n64, pl.BlockSpec((BN, NUM_RB), lambda i: (i, 0))],
            [jax.ShapeDtypeStruct((N, D), f32),
             jax.ShapeDtypeStruct((N, NUM_RB), f32)],
            (xi, accv, den, p["Ws"][l], _row(p["bs"][l]), _row(p["g1"][l]),
             _row(p["be1"][l]), p["W1"][l], _row(p["b1"][l]), p["W2"][l],
             _row(p["b2"][l]), _row(p["g2"][l]), _row(p["be2"][l]),
             p["W_actor"], _row(p["b_actor"])),
        )

    sums, cnts, val = _tc_call(
        _pool_body, (nb,),
        [spec_n64, pl.BlockSpec((BN, 1), lambda i: (i, 0)),
         spec_w(D, 1), spec_w(1, 1)],
        [pl.BlockSpec((G, D), lambda i: (0, 0)),
         pl.BlockSpec((G, D), lambda i: (0, 0)),
         pl.BlockSpec((G, 1), lambda i: (0, 0))],
        [jax.ShapeDtypeStruct((G, D), f32), jax.ShapeDtypeStruct((G, D), f32),
         jax.ShapeDtypeStruct((G, 1), f32)],
        (x, batch2, p["W_critic"], p["b_critic"].reshape(1, 1)),
    )
    value = val[:, 0]
    return x, value, logit


# speedup vs baseline: 1.0350x; 1.0350x over previous
import functools

import jax
import jax.numpy as jnp
from jax import lax
from jax.experimental import pallas as pl
from jax.experimental.pallas import tpu as pltpu
from jax.experimental.pallas import tpu_sc as plsc

N = 50000
E = 800000
G = 8
D = 64
H = 4
OC = D // H
DFF = 256
L = 2
NUM_RB = 4

NC = 2
NS = 16
NW = NC * NS

EPAD = 819200
GB = 1024
PW_G = EPAD // NW
GBG = 256
NCH2 = PW_G // GBG // 2
PW_S = EPAD // NS
NCH_S = PW_S // GB

NWIN = 12544
NROUND = 2
ACCRV = 12672
ACCRX = 6400
ZRV = ACCRV // NS
ZRX = ACCRX // NS

BN = 2000
BE = 2048


def _prologue_body(pa_ref, ba_ref, npa_ref, wpa_ref, wba_ref, bin_ref,
                   wemb_ref, bemb_ref, inp_ref, x0_ref):
    inp_ref[...] = (pa_ref[...] @ wpa_ref[...] + ba_ref[...] @ wba_ref[...]
                    + bin_ref[...])
    x0_ref[...] = npa_ref[...] @ wemb_ref[...] + bemb_ref[...]


def _qkv_body(x_ref, inp_ref, wq_ref, bq_ref, wk_ref, bk_ref, wv_ref, bv_ref,
              xi_ref, q_ref, kv_ref):
    xi = x_ref[...] + inp_ref[...]
    xi_ref[...] = xi
    q_ref[...] = xi @ wq_ref[...] + bq_ref[...]
    kv_ref[...] = jnp.concatenate(
        [xi @ wk_ref[...] + bk_ref[...], xi @ wv_ref[...] + bv_ref[...]],
        axis=1)


def _edge_body(qd_ref, kvs_ref, a_ref, dst_ref, we_ref, ov_ref, ox_ref):
    e = a_ref[...] @ we_ref[...]
    qd = qd_ref[...]
    kvs = kvs_ref[...]
    ke = kvs[:, :D] + e
    ve = kvs[:, D:] + e
    parts = []
    exs = []
    for h in range(H):
        sl = slice(h * OC, (h + 1) * OC)
        alpha = jnp.sum(qd[:, sl] * ke[:, sl], axis=1, keepdims=True) * 0.25
        ex = jnp.exp(alpha)
        parts.append(ve[:, sl] * ex)
        exs.append(ex)
    ov_ref[...] = jnp.concatenate(parts, axis=1)
    ex4 = jnp.concatenate(exs, axis=1)
    odd = (dst_ref[...] % 2) == 1
    zero = jnp.zeros_like(ex4)
    ox_ref[...] = jnp.concatenate(
        [jnp.where(odd, zero, ex4), jnp.where(odd, ex4, zero)], axis=1)


def _layer_norm(x, g, b):
    m = jnp.mean(x, axis=-1, keepdims=True)
    v = jnp.mean((x - m) ** 2, axis=-1, keepdims=True)
    return (x - m) * jax.lax.rsqrt(v + 1e-5) * g + b


def _node_body(xi_ref, accv_ref, den_ref, ws_ref, bs_ref, g1_ref, be1_ref,
               w1_ref, b1_ref, w2_ref, b2_ref, g2_ref, be2_ref, wa_ref,
               ba_ref, xn_ref, logit_ref):
    acc = accv_ref[...]
    dens = den_ref[...]
    aggs = []
    for h in range(H):
        den = dens[:, h:h + 1] + 1e-16
        aggs.append(acc[:, h * OC:(h + 1) * OC] / den)
    agg = jnp.concatenate(aggs, axis=1)
    xi = xi_ref[...]
    x2 = agg + xi @ ws_ref[...] + bs_ref[...]
    y = _layer_norm(xi + x2, g1_ref[...], be1_ref[...])
    z = jax.nn.relu(y @ w1_ref[...] + b1_ref[...]) @ w2_ref[...] + b2_ref[...]
    xn = _layer_norm(y + z, g2_ref[...], be2_ref[...])
    xn_ref[...] = xn
    logit_ref[...] = xn @ wa_ref[...] + ba_ref[...]


def _pool_body(x_ref, b_ref, wc_ref, bc_ref, sums_ref, cnts_ref, val_ref):
    i = pl.program_id(0)

    @pl.when(i == 0)
    def _init():
        sums_ref[...] = jnp.zeros_like(sums_ref)
        cnts_ref[...] = jnp.zeros_like(cnts_ref)

    x = x_ref[...]
    bv = b_ref[...]
    rows_s = []
    rows_c = []
    for g in range(G):
        mask = (bv == g).astype(jnp.float32)
        rows_s.append(jnp.sum(mask * x, axis=0, keepdims=True))
        rows_c.append(jnp.sum(mask, axis=0, keepdims=True) *
                      jnp.ones((1, D), jnp.float32))
    sums_ref[...] += jnp.concatenate(rows_s, axis=0)
    cnts_ref[...] += jnp.concatenate(rows_c, axis=0)

    @pl.when(i == pl.num_programs(0) - 1)
    def _fin():
        pooled = sums_ref[...] / jnp.maximum(cnts_ref[...], 1.0)
        val_ref[...] = pooled @ wc_ref[...] + bc_ref[...]


_MESH = plsc.VectorSubcoreMesh(core_axis_name="c", subcore_axis_name="s",
                               num_cores=NC, num_subcores=NS)


@functools.partial(
    pl.kernel,
    out_type=[jax.ShapeDtypeStruct((EPAD, 2 * D), jnp.float32),
              jax.ShapeDtypeStruct((EPAD, D), jnp.float32)],
    mesh=_MESH,
    compiler_params=pltpu.CompilerParams(use_tc_tiling_on_sc=False),
    scratch_types=[
        pltpu.VMEM((2, 256), jnp.int32),
        pltpu.VMEM((2, 256), jnp.int32),
        pltpu.VMEM((GBG, 2 * D), jnp.float32),
        pltpu.VMEM((GBG, 2 * D), jnp.float32),
        pltpu.VMEM((GBG, D), jnp.float32),
        pltpu.VMEM((GBG, D), jnp.float32),
        pltpu.SemaphoreType.DMA,
        pltpu.SemaphoreType.DMA,
        pltpu.SemaphoreType.DMA,
        pltpu.SemaphoreType.DMA,
        pltpu.SemaphoreType.DMA,
        pltpu.SemaphoreType.DMA,
    ],
)
def _sc_gather(sd_hbm, kv_hbm, q_hbm, kvs_hbm, qd_hbm,
               ib0, ib1, kvb0, kvb1, qdb0, qdb1,
               isem0, isem1, gsem0, gsem1, wsem0, wsem1):
    c = lax.axis_index("c")
    s = lax.axis_index("s")
    wid = c * NS + s
    cbase = wid * (2 * NCH2)

    bufs = ((ib0, kvb0, qdb0, isem0, gsem0, wsem0),
            (ib1, kvb1, qdb1, isem1, gsem1, wsem1))

    for half, (ib, kvb, qdb, isem, gsem, wsem) in enumerate(bufs):
        r0 = pl.multiple_of((cbase + half) * 2, 2)
        pltpu.async_copy(sd_hbm.at[pl.ds(r0, 2)], ib, isem)

    def body(i2, _):
        for half, (ib, kvb, qdb, isem, gsem, wsem) in enumerate(bufs):
            ci = cbase + 2 * i2 + half
            off = pl.multiple_of(ci * GBG, GBG)
            r0 = pl.multiple_of(ci * 2, 2)
            pltpu.make_async_copy(sd_hbm.at[pl.ds(r0, 2)], ib, isem).wait()

            @pl.when(i2 > 0)
            def _wait_writes():
                poff = pl.multiple_of(off - 2 * GBG, GBG)
                pltpu.make_async_copy(
                    kvb, kvs_hbm.at[pl.ds(poff, GBG)], wsem).wait()
                pltpu.make_async_copy(
                    qdb, qd_hbm.at[pl.ds(poff, GBG)], wsem).wait()

            g = [pltpu.async_copy(kv_hbm.at[ib.at[0]], kvb, gsem),
                 pltpu.async_copy(q_hbm.at[ib.at[1]], qdb, gsem)]
            for d in g:
                d.wait()

            @pl.when(i2 < NCH2 - 1)
            def _prefetch():
                r2 = pl.multiple_of((ci + 2) * 2, 2)
                pltpu.async_copy(sd_hbm.at[pl.ds(r2, 2)], ib, isem)

            pltpu.async_copy(kvb, kvs_hbm.at[pl.ds(off, GBG)], wsem)
            pltpu.async_copy(qdb, qd_hbm.at[pl.ds(off, GBG)], wsem)
        return 0

    lax.fori_loop(0, NCH2, body, 0)

    for half, (ib, kvb, qdb, isem, gsem, wsem) in enumerate(bufs):
        loff = pl.multiple_of((cbase + 2 * (NCH2 - 1) + half) * GBG, GBG)
        pltpu.make_async_copy(kvb, kvs_hbm.at[pl.ds(loff, GBG)], wsem).wait()
        pltpu.make_async_copy(qdb, qd_hbm.at[pl.ds(loff, GBG)], wsem).wait()


@functools.partial(
    pl.kernel,
    out_type=[jax.ShapeDtypeStruct((2 * NROUND * ACCRV, D), jnp.float32),
              jax.ShapeDtypeStruct((2 * NROUND * ACCRX, 8), jnp.float32)],
    mesh=_MESH,
    compiler_params=pltpu.CompilerParams(use_tc_tiling_on_sc=False),
    scratch_types=[
        pltpu.VMEM((8, 128), jnp.int32),
        pltpu.VMEM((2, 512), jnp.int32),
        pltpu.VMEM((2, 512), jnp.int32),
        pltpu.VMEM((GB, D), jnp.float32),
        pltpu.VMEM((GB, 8), jnp.float32),
        pltpu.VMEM_SHARED((ACCRV, D), jnp.float32),
        pltpu.VMEM_SHARED((ACCRX, 8), jnp.float32),
    ],
)
def _sc_scatter(dst2_hbm, msgv_hbm, msgx_hbm, zerov_hbm, zerox_hbm,
                outv_hbm, outx_hbm,
                didx_v, lidxv_v, lidxx_v, rowsv_v, rowsx_v, accv_sh, accx_sh):
    c = lax.axis_index("c")
    s = lax.axis_index("s")
    lanes = lax.iota(jnp.int32, 16)

    def do_round(r, _):
        w = 2 * r + c
        nbase = w * NWIN

        pltpu.sync_copy(zerov_hbm, accv_sh.at[pl.ds(s * ZRV, ZRV)])
        pltpu.sync_copy(zerox_hbm, accx_sh.at[pl.ds(s * ZRX, ZRX)])
        plsc.subcore_barrier()

        def chunk(i, _):
            off = pl.multiple_of(s * PW_S + i * GB, GB)
            row0 = pl.multiple_of(off // 128, 8)
            pltpu.sync_copy(dst2_hbm.at[pl.ds(row0, 8)], didx_v)
            pltpu.sync_copy(msgv_hbm.at[pl.ds(off, GB)], rowsv_v)
            pltpu.sync_copy(msgx_hbm.at[pl.ds(off, GB)], rowsx_v)
            for j in range(8):
                for t in range(8):
                    dvec = didx_v[j, pl.ds(t * 16, 16)]
                    lvec = dvec - nbase
                    gpos = off + (j * 128 + t * 16) + lanes
                    ok = (lvec >= 0) & (lvec < NWIN) & (gpos < E)
                    tv = NWIN + (dvec & 127)
                    tx = (NWIN // 2) + (dvec & 127)
                    q, rr = divmod(j * 128 + t * 16, 512)
                    lidxv_v[q, pl.ds(rr, 16)] = jnp.where(ok, lvec, tv)
                    lidxx_v[q, pl.ds(rr, 16)] = jnp.where(ok, lvec >> 1, tx)
            for j in range(2):
                pltpu.sync_copy(rowsv_v.at[pl.ds(j * 512, 512)],
                                accv_sh.at[lidxv_v.at[j]], add=True)
                pltpu.sync_copy(rowsx_v.at[pl.ds(j * 512, 512)],
                                accx_sh.at[lidxx_v.at[j]], add=True)
            return 0

        lax.fori_loop(0, NCH_S, chunk, 0)

        plsc.subcore_barrier()
        pltpu.sync_copy(accv_sh.at[pl.ds(s * ZRV, ZRV)],
                        outv_hbm.at[pl.ds(w * ACCRV + s * ZRV, ZRV)])
        pltpu.sync_copy(accx_sh.at[pl.ds(s * ZRX, ZRX)],
                        outx_hbm.at[pl.ds(w * ACCRX + s * ZRX, ZRX)])
        plsc.subcore_barrier()
        return 0

    lax.fori_loop(0, NROUND, do_round, 0)


def _tc_call(body, grid, in_specs, out_specs, out_shape, args):
    return pl.pallas_call(
        body,
        grid=grid,
        in_specs=in_specs,
        out_specs=out_specs,
        out_shape=out_shape,
    )(*args)


def _row(x):
    return x.reshape(1, -1)


def kernel(power_alloc, beam_alloc, node_power_attn, edge_power_attn,
           edge_index, ptr, batch, params):
    p = params
    f32 = jnp.float32

    pa = power_alloc.reshape(N, 16)
    ba = beam_alloc.reshape(N, 8)
    npa = node_power_attn.reshape(N, -1)
    ea = edge_power_attn.reshape(E, -1)
    ea_p = jnp.concatenate([ea, jnp.zeros((EPAD - E, 16), f32)], axis=0)
    src_p = jnp.concatenate([edge_index[0], jnp.zeros((EPAD - E,), jnp.int32)])
    dst_p = jnp.concatenate([edge_index[1], jnp.zeros((EPAD - E,), jnp.int32)])
    dst2 = dst_p.reshape(EPAD // 128, 128)
    sd = jnp.concatenate([src_p.reshape(EPAD // GBG, 1, GBG),
                          dst_p.reshape(EPAD // GBG, 1, GBG)],
                         axis=1).reshape(EPAD // GBG * 2, GBG)
    batch2 = batch.reshape(N, 1)
    dstcol = dst_p.reshape(EPAD, 1)
    zerov = jnp.zeros((ZRV, D), f32)
    zerox = jnp.zeros((ZRX, 8), f32)

    w_in = p["W_in"]
    idx_pa = [6 * rb + j for rb in range(NUM_RB) for j in range(4)]
    idx_ba = [6 * rb + 4 + j for rb in range(NUM_RB) for j in range(2)]
    wpa = w_in[jnp.array(idx_pa)]
    wba = w_in[jnp.array(idx_ba)]

    nb = N // BN
    spec_n64 = pl.BlockSpec((BN, D), lambda i: (i, 0))
    spec_w = lambda r, c: pl.BlockSpec((r, c), lambda i: (0, 0))

    inp, x = _tc_call(
        _prologue_body, (nb,),
        [pl.BlockSpec((BN, 16), lambda i: (i, 0)),
         pl.BlockSpec((BN, 8), lambda i: (i, 0)),
         pl.BlockSpec((BN, 16), lambda i: (i, 0)),
         spec_w(16, D), spec_w(8, D), spec_w(1, D),
         spec_w(16, D), spec_w(1, D)],
        [spec_n64, spec_n64],
        [jax.ShapeDtypeStruct((N, D), f32)] * 2,
        (pa, ba, npa, wpa, wba, _row(p["b_in"]), p["W_emb"], _row(p["b_emb"])),
    )

    logit = None
    for l in range(L):
        xi, q_t, kv_t = _tc_call(
            _qkv_body, (nb,),
            [spec_n64, spec_n64,
             spec_w(D, D), spec_w(1, D), spec_w(D, D), spec_w(1, D),
             spec_w(D, D), spec_w(1, D)],
            [spec_n64, spec_n64, pl.BlockSpec((BN, 2 * D), lambda i: (i, 0))],
            [jax.ShapeDtypeStruct((N, D), f32),
             jax.ShapeDtypeStruct((N, D), f32),
             jax.ShapeDtypeStruct((N, 2 * D), f32)],
            (x, inp, p["Wq"][l], _row(p["bq"][l]), p["Wk"][l],
             _row(p["bk"][l]), p["Wv"][l], _row(p["bv"][l])),
        )

        kvs, qd = _sc_gather(sd, kv_t, q_t)

        neb = EPAD // BE
        spec_e64 = pl.BlockSpec((BE, D), lambda i: (i, 0))
        msgv, msgx = _tc_call(
            _edge_body, (neb,),
            [spec_e64, pl.BlockSpec((BE, 2 * D), lambda i: (i, 0)),
             pl.BlockSpec((BE, 16), lambda i: (i, 0)),
             pl.BlockSpec((BE, 1), lambda i: (i, 0)),
             pl.BlockSpec((16, D), lambda i: (0, 0))],
            [spec_e64, pl.BlockSpec((BE, 8), lambda i: (i, 0))],
            [jax.ShapeDtypeStruct((EPAD, D), f32),
             jax.ShapeDtypeStruct((EPAD, 8), f32)],
            (qd, kvs, ea_p, dstcol, p["We"][l]),
        )

        accv_pad, accx_pad = _sc_scatter(dst2, msgv, msgx, zerov, zerox)
        accv = jnp.concatenate(
            [accv_pad[w * ACCRV:w * ACCRV + NWIN] for w in range(2 * NROUND)],
            axis=0)[:N]
        den = jnp.concatenate(
            [accx_pad[w * ACCRX:w * ACCRX + NWIN // 2].reshape(NWIN, 4)
             for w in range(2 * NROUND)], axis=0)[:N]

        x, logit = _tc_call(
            _node_body, (nb,),
            [spec_n64, spec_n64, pl.BlockSpec((BN, 4), lambda i: (i, 0)),
             spec_w(D, D), spec_w(1, D), spec_w(1, D), spec_w(1, D),
             spec_w(D, DFF), spec_w(1, DFF), spec_w(DFF, D), spec_w(1, D),
             spec_w(1, D), spec_w(1, D), spec_w(D, NUM_RB), spec_w(1, NUM_RB)],
            [spec_n64, pl.BlockSpec((BN, NUM_RB), lambda i: (i, 0))],
            [jax.ShapeDtypeStruct((N, D), f32),
             jax.ShapeDtypeStruct((N, NUM_RB), f32)],
            (xi, accv, den, p["Ws"][l], _row(p["bs"][l]), _row(p["g1"][l]),
             _row(p["be1"][l]), p["W1"][l], _row(p["b1"][l]), p["W2"][l],
             _row(p["b2"][l]), _row(p["g2"][l]), _row(p["be2"][l]),
             p["W_actor"], _row(p["b_actor"])),
        )

    sums, cnts, val = _tc_call(
        _pool_body, (nb,),
        [spec_n64, pl.BlockSpec((BN, 1), lambda i: (i, 0)),
         spec_w(D, 1), spec_w(1, 1)],
        [pl.BlockSpec((G, D), lambda i: (0, 0)),
         pl.BlockSpec((G, D), lambda i: (0, 0)),
         pl.BlockSpec((G, 1), lambda i: (0, 0))],
        [jax.ShapeDtypeStruct((G, D), f32), jax.ShapeDtypeStruct((G, D), f32),
         jax.ShapeDtypeStruct((G, 1), f32)],
        (x, batch2, p["W_critic"], p["b_critic"].reshape(1, 1)),
    )
    value = val[:, 0]
    return x, value, logit


# speedup vs baseline: 1.0872x; 1.0505x over previous
import functools

import jax
import jax.numpy as jnp
from jax import lax
from jax.experimental import pallas as pl
from jax.experimental.pallas import tpu as pltpu
from jax.experimental.pallas import tpu_sc as plsc

N = 50000
E = 800000
G = 8
D = 64
H = 4
OC = D // H
DFF = 256
L = 2
NUM_RB = 4

NC = 2
NS = 16
NW = NC * NS

EPAD = 819200
GB = 1024
PW_G = EPAD // NW
GBG = 256
NCH2 = PW_G // GBG // 2
PW_S = EPAD // NS
GBS = 512
NCHS2 = PW_S // GBS // 2

NWIN = 12544
NROUND = 2
ACCRV = 12672
ACCRX = 6400
ZRV = ACCRV // NS
ZRX = ACCRX // NS

BN = 2000
BE = 2048


def _prologue_body(pa_ref, ba_ref, npa_ref, wpa_ref, wba_ref, bin_ref,
                   wemb_ref, bemb_ref, inp_ref, x0_ref):
    inp_ref[...] = (pa_ref[...] @ wpa_ref[...] + ba_ref[...] @ wba_ref[...]
                    + bin_ref[...])
    x0_ref[...] = npa_ref[...] @ wemb_ref[...] + bemb_ref[...]


def _qkv_body(x_ref, inp_ref, wq_ref, bq_ref, wk_ref, bk_ref, wv_ref, bv_ref,
              xi_ref, q_ref, kv_ref):
    xi = x_ref[...] + inp_ref[...]
    xi_ref[...] = xi
    q_ref[...] = xi @ wq_ref[...] + bq_ref[...]
    kv_ref[...] = jnp.concatenate(
        [xi @ wk_ref[...] + bk_ref[...], xi @ wv_ref[...] + bv_ref[...]],
        axis=1)


def _edge_body(qd_ref, kvs_ref, a_ref, dst_ref, we_ref, ov_ref, ox_ref):
    e = a_ref[...] @ we_ref[...]
    qd = qd_ref[...]
    kvs = kvs_ref[...]
    ke = kvs[:, :D] + e
    ve = kvs[:, D:] + e
    parts = []
    exs = []
    for h in range(H):
        sl = slice(h * OC, (h + 1) * OC)
        alpha = jnp.sum(qd[:, sl] * ke[:, sl], axis=1, keepdims=True) * 0.25
        ex = jnp.exp(alpha)
        parts.append(ve[:, sl] * ex)
        exs.append(ex)
    ov_ref[...] = jnp.concatenate(parts, axis=1)
    ex4 = jnp.concatenate(exs, axis=1)
    odd = (dst_ref[...] % 2) == 1
    zero = jnp.zeros_like(ex4)
    ox_ref[...] = jnp.concatenate(
        [jnp.where(odd, zero, ex4), jnp.where(odd, ex4, zero)], axis=1)


def _layer_norm(x, g, b):
    m = jnp.mean(x, axis=-1, keepdims=True)
    v = jnp.mean((x - m) ** 2, axis=-1, keepdims=True)
    return (x - m) * jax.lax.rsqrt(v + 1e-5) * g + b


def _node_body(xi_ref, accv_ref, den_ref, ws_ref, bs_ref, g1_ref, be1_ref,
               w1_ref, b1_ref, w2_ref, b2_ref, g2_ref, be2_ref, wa_ref,
               ba_ref, xn_ref, logit_ref):
    acc = accv_ref[...]
    dens = den_ref[...]
    aggs = []
    for h in range(H):
        den = dens[:, h:h + 1] + 1e-16
        aggs.append(acc[:, h * OC:(h + 1) * OC] / den)
    agg = jnp.concatenate(aggs, axis=1)
    xi = xi_ref[...]
    x2 = agg + xi @ ws_ref[...] + bs_ref[...]
    y = _layer_norm(xi + x2, g1_ref[...], be1_ref[...])
    z = jax.nn.relu(y @ w1_ref[...] + b1_ref[...]) @ w2_ref[...] + b2_ref[...]
    xn = _layer_norm(y + z, g2_ref[...], be2_ref[...])
    xn_ref[...] = xn
    logit_ref[...] = xn @ wa_ref[...] + ba_ref[...]


def _pool_body(x_ref, b_ref, wc_ref, bc_ref, sums_ref, cnts_ref, val_ref):
    i = pl.program_id(0)

    @pl.when(i == 0)
    def _init():
        sums_ref[...] = jnp.zeros_like(sums_ref)
        cnts_ref[...] = jnp.zeros_like(cnts_ref)

    x = x_ref[...]
    bv = b_ref[...]
    rows_s = []
    rows_c = []
    for g in range(G):
        mask = (bv == g).astype(jnp.float32)
        rows_s.append(jnp.sum(mask * x, axis=0, keepdims=True))
        rows_c.append(jnp.sum(mask, axis=0, keepdims=True) *
                      jnp.ones((1, D), jnp.float32))
    sums_ref[...] += jnp.concatenate(rows_s, axis=0)
    cnts_ref[...] += jnp.concatenate(rows_c, axis=0)

    @pl.when(i == pl.num_programs(0) - 1)
    def _fin():
        pooled = sums_ref[...] / jnp.maximum(cnts_ref[...], 1.0)
        val_ref[...] = pooled @ wc_ref[...] + bc_ref[...]


_MESH = plsc.VectorSubcoreMesh(core_axis_name="c", subcore_axis_name="s",
                               num_cores=NC, num_subcores=NS)


@functools.partial(
    pl.kernel,
    out_type=[jax.ShapeDtypeStruct((EPAD, 2 * D), jnp.float32),
              jax.ShapeDtypeStruct((EPAD, D), jnp.float32)],
    mesh=_MESH,
    compiler_params=pltpu.CompilerParams(use_tc_tiling_on_sc=False),
    scratch_types=[
        pltpu.VMEM((2, 256), jnp.int32),
        pltpu.VMEM((2, 256), jnp.int32),
        pltpu.VMEM((GBG, 2 * D), jnp.float32),
        pltpu.VMEM((GBG, 2 * D), jnp.float32),
        pltpu.VMEM((GBG, D), jnp.float32),
        pltpu.VMEM((GBG, D), jnp.float32),
        pltpu.SemaphoreType.DMA,
        pltpu.SemaphoreType.DMA,
        pltpu.SemaphoreType.DMA,
        pltpu.SemaphoreType.DMA,
        pltpu.SemaphoreType.DMA,
        pltpu.SemaphoreType.DMA,
    ],
)
def _sc_gather(sd_hbm, kv_hbm, q_hbm, kvs_hbm, qd_hbm,
               ib0, ib1, kvb0, kvb1, qdb0, qdb1,
               isem0, isem1, gsem0, gsem1, wsem0, wsem1):
    c = lax.axis_index("c")
    s = lax.axis_index("s")
    wid = c * NS + s
    cbase = wid * (2 * NCH2)

    bufs = ((ib0, kvb0, qdb0, isem0, gsem0, wsem0),
            (ib1, kvb1, qdb1, isem1, gsem1, wsem1))

    for half, (ib, kvb, qdb, isem, gsem, wsem) in enumerate(bufs):
        r0 = pl.multiple_of((cbase + half) * 2, 2)
        pltpu.async_copy(sd_hbm.at[pl.ds(r0, 2)], ib, isem)

    def body(i2, _):
        for half, (ib, kvb, qdb, isem, gsem, wsem) in enumerate(bufs):
            ci = cbase + 2 * i2 + half
            off = pl.multiple_of(ci * GBG, GBG)
            r0 = pl.multiple_of(ci * 2, 2)
            pltpu.make_async_copy(sd_hbm.at[pl.ds(r0, 2)], ib, isem).wait()

            @pl.when(i2 > 0)
            def _wait_writes():
                poff = pl.multiple_of(off - 2 * GBG, GBG)
                pltpu.make_async_copy(
                    kvb, kvs_hbm.at[pl.ds(poff, GBG)], wsem).wait()
                pltpu.make_async_copy(
                    qdb, qd_hbm.at[pl.ds(poff, GBG)], wsem).wait()

            g = [pltpu.async_copy(kv_hbm.at[ib.at[0]], kvb, gsem),
                 pltpu.async_copy(q_hbm.at[ib.at[1]], qdb, gsem)]
            for d in g:
                d.wait()

            @pl.when(i2 < NCH2 - 1)
            def _prefetch():
                r2 = pl.multiple_of((ci + 2) * 2, 2)
                pltpu.async_copy(sd_hbm.at[pl.ds(r2, 2)], ib, isem)

            pltpu.async_copy(kvb, kvs_hbm.at[pl.ds(off, GBG)], wsem)
            pltpu.async_copy(qdb, qd_hbm.at[pl.ds(off, GBG)], wsem)
        return 0

    lax.fori_loop(0, NCH2, body, 0)

    for half, (ib, kvb, qdb, isem, gsem, wsem) in enumerate(bufs):
        loff = pl.multiple_of((cbase + 2 * (NCH2 - 1) + half) * GBG, GBG)
        pltpu.make_async_copy(kvb, kvs_hbm.at[pl.ds(loff, GBG)], wsem).wait()
        pltpu.make_async_copy(qdb, qd_hbm.at[pl.ds(loff, GBG)], wsem).wait()


@functools.partial(
    pl.kernel,
    out_type=[jax.ShapeDtypeStruct((2 * NROUND * ACCRV, D), jnp.float32),
              jax.ShapeDtypeStruct((2 * NROUND * ACCRX, 8), jnp.float32)],
    mesh=_MESH,
    compiler_params=pltpu.CompilerParams(use_tc_tiling_on_sc=False),
    scratch_types=[
        pltpu.VMEM((4, 128), jnp.int32),
        pltpu.VMEM((4, 128), jnp.int32),
        pltpu.VMEM((1, GBS), jnp.int32),
        pltpu.VMEM((1, GBS), jnp.int32),
        pltpu.VMEM((GBS, D), jnp.float32),
        pltpu.VMEM((GBS, D), jnp.float32),
        pltpu.VMEM((GBS, 8), jnp.float32),
        pltpu.VMEM((GBS, 8), jnp.float32),
        pltpu.VMEM_SHARED((ACCRV, D), jnp.float32),
        pltpu.VMEM_SHARED((ACCRX, 8), jnp.float32),
        pltpu.SemaphoreType.DMA,
        pltpu.SemaphoreType.DMA,
    ],
)
def _sc_scatter(dst2_hbm, msgv_hbm, msgx_hbm, zerov_hbm, zerox_hbm,
                outv_hbm, outx_hbm,
                didx0, didx1, lidxv_v, lidxx_v, rv0, rv1, rx0, rx1,
                accv_sh, accx_sh, psem0, psem1):
    c = lax.axis_index("c")
    s = lax.axis_index("s")
    lanes = lax.iota(jnp.int32, 16)
    bufs = ((didx0, rv0, rx0, psem0), (didx1, rv1, rx1, psem1))

    def prefetch(ci, didx, rv, rx, psem):
        off = pl.multiple_of(s * PW_S + ci * GBS, GBS)
        row0 = pl.multiple_of(off // 128, 4)
        pltpu.async_copy(dst2_hbm.at[pl.ds(row0, 4)], didx, psem)
        pltpu.async_copy(msgv_hbm.at[pl.ds(off, GBS)], rv, psem)
        pltpu.async_copy(msgx_hbm.at[pl.ds(off, GBS)], rx, psem)

    def wait_prefetch(ci, didx, rv, rx, psem):
        off = pl.multiple_of(s * PW_S + ci * GBS, GBS)
        row0 = pl.multiple_of(off // 128, 4)
        pltpu.make_async_copy(dst2_hbm.at[pl.ds(row0, 4)], didx, psem).wait()
        pltpu.make_async_copy(msgv_hbm.at[pl.ds(off, GBS)], rv, psem).wait()
        pltpu.make_async_copy(msgx_hbm.at[pl.ds(off, GBS)], rx, psem).wait()

    def do_round(r, _):
        w = 2 * r + c
        nbase = w * NWIN

        pltpu.sync_copy(zerov_hbm, accv_sh.at[pl.ds(s * ZRV, ZRV)])
        pltpu.sync_copy(zerox_hbm, accx_sh.at[pl.ds(s * ZRX, ZRX)])
        plsc.subcore_barrier()

        for half, (didx, rv, rx, psem) in enumerate(bufs):
            prefetch(half, didx, rv, rx, psem)

        def step(i2, _):
            for half, (didx, rv, rx, psem) in enumerate(bufs):
                ci = 2 * i2 + half
                off = pl.multiple_of(s * PW_S + ci * GBS, GBS)
                wait_prefetch(ci, didx, rv, rx, psem)
                for j in range(4):
                    for t in range(8):
                        dvec = didx[j, pl.ds(t * 16, 16)]
                        lvec = dvec - nbase
                        gpos = off + (j * 128 + t * 16) + lanes
                        ok = (lvec >= 0) & (lvec < NWIN) & (gpos < E)
                        tv = NWIN + (dvec & 127)
                        tx = (NWIN // 2) + (dvec & 127)
                        pos = j * 128 + t * 16
                        lidxv_v[0, pl.ds(pos, 16)] = jnp.where(ok, lvec, tv)
                        lidxx_v[0, pl.ds(pos, 16)] = jnp.where(
                            ok, lvec >> 1, tx)
                pltpu.sync_copy(rv, accv_sh.at[lidxv_v.at[0]], add=True)
                pltpu.sync_copy(rx, accx_sh.at[lidxx_v.at[0]], add=True)

                @pl.when(i2 < NCHS2 - 1)
                def _pf():
                    prefetch(ci + 2, didx, rv, rx, psem)
            return 0

        lax.fori_loop(0, NCHS2, step, 0)

        plsc.subcore_barrier()
        pltpu.sync_copy(accv_sh.at[pl.ds(s * ZRV, ZRV)],
                        outv_hbm.at[pl.ds(w * ACCRV + s * ZRV, ZRV)])
        pltpu.sync_copy(accx_sh.at[pl.ds(s * ZRX, ZRX)],
                        outx_hbm.at[pl.ds(w * ACCRX + s * ZRX, ZRX)])
        plsc.subcore_barrier()
        return 0

    lax.fori_loop(0, NROUND, do_round, 0)


def _tc_call(body, grid, in_specs, out_specs, out_shape, args):
    return pl.pallas_call(
        body,
        grid=grid,
        in_specs=in_specs,
        out_specs=out_specs,
        out_shape=out_shape,
    )(*args)


def _row(x):
    return x.reshape(1, -1)


def kernel(power_alloc, beam_alloc, node_power_attn, edge_power_attn,
           edge_index, ptr, batch, params):
    p = params
    f32 = jnp.float32

    pa = power_alloc.reshape(N, 16)
    ba = beam_alloc.reshape(N, 8)
    npa = node_power_attn.reshape(N, -1)
    ea = edge_power_attn.reshape(E, -1)
    ea_p = jnp.concatenate([ea, jnp.zeros((EPAD - E, 16), f32)], axis=0)
    src_p = jnp.concatenate([edge_index[0], jnp.zeros((EPAD - E,), jnp.int32)])
    dst_p = jnp.concatenate([edge_index[1], jnp.zeros((EPAD - E,), jnp.int32)])
    dst2 = dst_p.reshape(EPAD // 128, 128)
    sd = jnp.concatenate([src_p.reshape(EPAD // GBG, 1, GBG),
                          dst_p.reshape(EPAD // GBG, 1, GBG)],
                         axis=1).reshape(EPAD // GBG * 2, GBG)
    batch2 = batch.reshape(N, 1)
    dstcol = dst_p.reshape(EPAD, 1)
    zerov = jnp.zeros((ZRV, D), f32)
    zerox = jnp.zeros((ZRX, 8), f32)

    w_in = p["W_in"]
    idx_pa = [6 * rb + j for rb in range(NUM_RB) for j in range(4)]
    idx_ba = [6 * rb + 4 + j for rb in range(NUM_RB) for j in range(2)]
    wpa = w_in[jnp.array(idx_pa)]
    wba = w_in[jnp.array(idx_ba)]

    nb = N // BN
    spec_n64 = pl.BlockSpec((BN, D), lambda i: (i, 0))
    spec_w = lambda r, c: pl.BlockSpec((r, c), lambda i: (0, 0))

    inp, x = _tc_call(
        _prologue_body, (nb,),
        [pl.BlockSpec((BN, 16), lambda i: (i, 0)),
         pl.BlockSpec((BN, 8), lambda i: (i, 0)),
         pl.BlockSpec((BN, 16), lambda i: (i, 0)),
         spec_w(16, D), spec_w(8, D), spec_w(1, D),
         spec_w(16, D), spec_w(1, D)],
        [spec_n64, spec_n64],
        [jax.ShapeDtypeStruct((N, D), f32)] * 2,
        (pa, ba, npa, wpa, wba, _row(p["b_in"]), p["W_emb"], _row(p["b_emb"])),
    )

    logit = None
    for l in range(L):
        xi, q_t, kv_t = _tc_call(
            _qkv_body, (nb,),
            [spec_n64, spec_n64,
             spec_w(D, D), spec_w(1, D), spec_w(D, D), spec_w(1, D),
             spec_w(D, D), spec_w(1, D)],
            [spec_n64, spec_n64, pl.BlockSpec((BN, 2 * D), lambda i: (i, 0))],
            [jax.ShapeDtypeStruct((N, D), f32),
             jax.ShapeDtypeStruct((N, D), f32),
             jax.ShapeDtypeStruct((N, 2 * D), f32)],
            (x, inp, p["Wq"][l], _row(p["bq"][l]), p["Wk"][l],
             _row(p["bk"][l]), p["Wv"][l], _row(p["bv"][l])),
        )

        kvs, qd = _sc_gather(sd, kv_t, q_t)

        neb = EPAD // BE
        spec_e64 = pl.BlockSpec((BE, D), lambda i: (i, 0))
        msgv, msgx = _tc_call(
            _edge_body, (neb,),
            [spec_e64, pl.BlockSpec((BE, 2 * D), lambda i: (i, 0)),
             pl.BlockSpec((BE, 16), lambda i: (i, 0)),
             pl.BlockSpec((BE, 1), lambda i: (i, 0)),
             pl.BlockSpec((16, D), lambda i: (0, 0))],
            [spec_e64, pl.BlockSpec((BE, 8), lambda i: (i, 0))],
            [jax.ShapeDtypeStruct((EPAD, D), f32),
             jax.ShapeDtypeStruct((EPAD, 8), f32)],
            (qd, kvs, ea_p, dstcol, p["We"][l]),
        )

        accv_pad, accx_pad = _sc_scatter(dst2, msgv, msgx, zerov, zerox)
        accv = jnp.concatenate(
            [accv_pad[w * ACCRV:w * ACCRV + NWIN] for w in range(2 * NROUND)],
            axis=0)[:N]
        den = jnp.concatenate(
            [accx_pad[w * ACCRX:w * ACCRX + NWIN // 2].reshape(NWIN, 4)
             for w in range(2 * NROUND)], axis=0)[:N]

        x, logit = _tc_call(
            _node_body, (nb,),
            [spec_n64, spec_n64, pl.BlockSpec((BN, 4), lambda i: (i, 0)),
             spec_w(D, D), spec_w(1, D), spec_w(1, D), spec_w(1, D),
             spec_w(D, DFF), spec_w(1, DFF), spec_w(DFF, D), spec_w(1, D),
             spec_w(1, D), spec_w(1, D), spec_w(D, NUM_RB), spec_w(1, NUM_RB)],
            [spec_n64, pl.BlockSpec((BN, NUM_RB), lambda i: (i, 0))],
            [jax.ShapeDtypeStruct((N, D), f32),
             jax.ShapeDtypeStruct((N, NUM_RB), f32)],
            (xi, accv, den, p["Ws"][l], _row(p["bs"][l]), _row(p["g1"][l]),
             _row(p["be1"][l]), p["W1"][l], _row(p["b1"][l]), p["W2"][l],
             _row(p["b2"][l]), _row(p["g2"][l]), _row(p["be2"][l]),
             p["W_actor"], _row(p["b_actor"])),
        )

    sums, cnts, val = _tc_call(
        _pool_body, (nb,),
        [spec_n64, pl.BlockSpec((BN, 1), lambda i: (i, 0)),
         spec_w(D, 1), spec_w(1, 1)],
        [pl.BlockSpec((G, D), lambda i: (0, 0)),
         pl.BlockSpec((G, D), lambda i: (0, 0)),
         pl.BlockSpec((G, 1), lambda i: (0, 0))],
        [jax.ShapeDtypeStruct((G, D), f32), jax.ShapeDtypeStruct((G, D), f32),
         jax.ShapeDtypeStruct((G, 1), f32)],
        (x, batch2, p["W_critic"], p["b_critic"].reshape(1, 1)),
    )
    value = val[:, 0]
    return x, value, logit
